# baseline (speedup 1.0000x reference)
import functools

import jax
import jax.numpy as jnp
from jax import lax
from jax.experimental import pallas as pl
from jax.experimental.pallas import tpu as pltpu
from jax.experimental.pallas import tpu_sc as plsc

N_FIELDS = 26
VOCAB = 100000
DIM = 32
BATCH = 4096

NUM_CORES = 2
NUM_SUBCORES = 16
LANES = 16
NW = NUM_CORES * NUM_SUBCORES

ROWS = BATCH * N_FIELDS
ROWS_W = ROWS // NW
GATHER_CHUNK = 128
N_CHUNKS = ROWS_W // GATHER_CHUNK


def _body(tab_hbm, idx_hbm, out_hbm, idx_v, rows_v, sem):
    wid = lax.axis_index("s") * NUM_CORES + lax.axis_index("c")
    base = wid * ROWS_W

    pltpu.sync_copy(idx_hbm.at[pl.ds(base, ROWS_W)], idx_v)

    lane = lax.iota(jnp.int32, LANES)

    def add_offsets(i, _):
        pos = lane + i * LANES
        f = lax.rem(pos, N_FIELDS)
        sl = pl.ds(i * LANES, LANES)
        idx_v[sl] = idx_v[sl] + f * VOCAB
        return _

    lax.fori_loop(0, ROWS_W // LANES, add_offsets, None)

    group = 13
    for g in range(N_CHUNKS // group):
        copies = []
        for j in range(g * group, (g + 1) * group):
            sl = pl.ds(j * GATHER_CHUNK, GATHER_CHUNK)
            copies.append(
                pltpu.make_async_copy(tab_hbm.at[idx_v.at[sl]], rows_v.at[sl], sem)
            )
        for c in copies:
            c.start()
        for c in copies:
            c.wait()

    pltpu.sync_copy(rows_v, out_hbm.at[pl.ds(base, ROWS_W)])


@jax.jit
def _gather(tab_flat, idx_flat):
    mesh = plsc.VectorSubcoreMesh(
        core_axis_name="c",
        subcore_axis_name="s",
        num_cores=NUM_CORES,
        num_subcores=NUM_SUBCORES,
    )
    return pl.kernel(
        _body,
        out_type=jax.ShapeDtypeStruct((ROWS, DIM), jnp.float32),
        mesh=mesh,
        scratch_types=[
            pltpu.VMEM((ROWS_W,), jnp.int32),
            pltpu.VMEM((ROWS_W, DIM), jnp.float32),
            pltpu.SemaphoreType.DMA,
        ],
        compiler_params=pltpu.CompilerParams(use_tc_tiling_on_sc=False),
    )(tab_flat, idx_flat)


def kernel(items, tables):
    tab_flat = tables.reshape(N_FIELDS * VOCAB, DIM)
    idx_flat = items.reshape(ROWS)
    out = _gather(tab_flat, idx_flat)
    return out.reshape(BATCH, N_FIELDS * DIM)
